# baseline (speedup 1.0000x reference)
import functools

import jax
import jax.numpy as jnp
from jax import lax
from jax.experimental import pallas as pl
from jax.experimental.pallas import tpu as pltpu
from jax.experimental.pallas import tpu_sc as plsc

_B = 8
_N = 4096
_D = 256
_S = 512
_K = 32
_R2 = 0.2 ** 2

_NC = 2
_NS = 16
_NW = _NC * _NS
_L = 16


_P = 4
_SC = _S // _P


def _fps_chunk_body(xyz_ref, dist_ref, farth_ref,
                    out_ref, dist_out_ref, farth_out_ref):
    x = xyz_ref[0]
    y = xyz_ref[1]
    z = xyz_ref[2]
    lane_iota = lax.broadcasted_iota(jnp.int32, (_B, _N), 1)
    s_iota = lax.broadcasted_iota(jnp.int32, (_B, _SC), 1)

    def body(k, carry):
        distance, farth, sx, sy, sz = carry
        onehot = lane_iota == farth
        cx = jnp.sum(jnp.where(onehot, x, 0.0), axis=1, keepdims=True)
        cy = jnp.sum(jnp.where(onehot, y, 0.0), axis=1, keepdims=True)
        cz = jnp.sum(jnp.where(onehot, z, 0.0), axis=1, keepdims=True)
        sel = s_iota == k
        sx = jnp.where(sel, cx, sx)
        sy = jnp.where(sel, cy, sy)
        sz = jnp.where(sel, cz, sz)
        dx = x - cx
        dy = y - cy
        dz = z - cz
        dist = dx * dx + dy * dy + dz * dz
        distance = jnp.minimum(dist, distance)
        m = jnp.max(distance, axis=1, keepdims=True)
        cand = jnp.where(distance == m, lane_iota, _N)
        farth = jnp.min(cand, axis=1, keepdims=True)
        return distance, farth, sx, sy, sz

    init = (
        dist_ref[...],
        farth_ref[...],
        jnp.zeros((_B, _SC), jnp.float32),
        jnp.zeros((_B, _SC), jnp.float32),
        jnp.zeros((_B, _SC), jnp.float32),
    )
    distance, farth, sx, sy, sz = lax.fori_loop(0, _SC, body, init)
    out_ref[0] = sx
    out_ref[1] = sy
    out_ref[2] = sz
    dist_out_ref[...] = distance
    farth_out_ref[...] = farth


_fps_chunk = pl.pallas_call(
    _fps_chunk_body,
    out_shape=(
        jax.ShapeDtypeStruct((3, _B, _SC), jnp.float32),
        jax.ShapeDtypeStruct((_B, _N), jnp.float32),
        jax.ShapeDtypeStruct((_B, 1), jnp.int32),
    ),
)


_SBLK = _SC // 4
_NCHUNK = _N // _L


_CBLK = 8


def _ballq_body(x_hbm, y_hbm, z_hbm, sx_hbm, sy_hbm, sz_hbm, out_hbm,
                xv, yv, zv, sxv, syv, szv, idxbuf, outv, cnt_ref):
    wid = lax.axis_index("s") * _NC + lax.axis_index("c")
    b = wid // 4
    q = wid % 4
    pltpu.sync_copy(x_hbm.at[pl.ds(b * _N, _N)], xv)
    pltpu.sync_copy(y_hbm.at[pl.ds(b * _N, _N)], yv)
    pltpu.sync_copy(z_hbm.at[pl.ds(b * _N, _N)], zv)
    soff = b * _SC + q * _SBLK
    pltpu.sync_copy(sx_hbm.at[pl.ds(soff, _SBLK)], sxv)
    pltpu.sync_copy(sy_hbm.at[pl.ds(soff, _SBLK)], syv)
    pltpu.sync_copy(sz_hbm.at[pl.ds(soff, _SBLK)], szv)

    r2 = jnp.float32(_R2)
    base_flat = b * _N
    i16 = lax.iota(jnp.int32, 16)

    def per_chunk16(t, _):
        svx = sxv[pl.ds(t * _L, _L)]
        svy = syv[pl.ds(t * _L, _L)]
        svz = szv[pl.ds(t * _L, _L)]
        for j in range(_L):
            i = t * _L + j
            sx = svx[j]
            sy = svy[j]
            sz = svz[j]
            cnt_ref[0] = jnp.int32(0)

            def blk(bi, _):
                cnt0 = cnt_ref[0]

                @pl.when(cnt0 < _K)
                def _():
                    cnt = cnt0
                    for cc in range(_CBLK):
                        c = bi * _CBLK + cc
                        px = xv[pl.ds(c * _L, _L)]
                        py = yv[pl.ds(c * _L, _L)]
                        pz = zv[pl.ds(c * _L, _L)]
                        dx = px - sx
                        dy = py - sy
                        dz = pz - sz
                        d = (dx * dx + dy * dy) + dz * dz
                        msk = d <= r2
                        ivec = i16 + c * _L
                        plsc.store_compressed(idxbuf.at[pl.ds(cnt, _L)],
                                              ivec, mask=msk)
                        cnt = cnt + plsc.all_reduce_population_count(msk)[0]
                    cnt_ref[0] = cnt
                return 0

            lax.fori_loop(0, _NCHUNK // _CBLK, blk, 0)
            cnt = cnt_ref[0]
            v0 = idxbuf[pl.ds(0, _L)]
            v1 = idxbuf[pl.ds(_L, _L)]
            first = v0[0]
            o0 = jnp.where(i16 < cnt, v0, first) + base_flat
            o1 = jnp.where((i16 + _L) < cnt, v1, first) + base_flat
            outv[i, pl.ds(0, _L)] = o0
            outv[i, pl.ds(_L, _L)] = o1
        return 0

    lax.fori_loop(0, _SBLK // _L, per_chunk16, 0)
    pltpu.sync_copy(outv, out_hbm.at[pl.ds(b * _SC + q * _SBLK, _SBLK)])


@functools.cache
def _get_ballq():
  return pl.kernel(
    _ballq_body,
    out_type=jax.ShapeDtypeStruct((_B * _SC, _K), jnp.int32),
    mesh=plsc.VectorSubcoreMesh(core_axis_name="c", subcore_axis_name="s",
                                num_cores=_NC, num_subcores=_NS),
    compiler_params=pltpu.CompilerParams(needs_layout_passes=False),
    scratch_types=[
        pltpu.VMEM((_N,), jnp.float32),
        pltpu.VMEM((_N,), jnp.float32),
        pltpu.VMEM((_N,), jnp.float32),
        pltpu.VMEM((_SBLK,), jnp.float32),
        pltpu.VMEM((_SBLK,), jnp.float32),
        pltpu.VMEM((_SBLK,), jnp.float32),
        pltpu.VMEM((160,), jnp.int32),
        pltpu.VMEM((_SBLK, _K), jnp.int32),
        pltpu.SMEM((1,), jnp.int32),
    ],
  )


_ROWS = _B * _S * _K
_RPW = _ROWS // _NW
_CHUNK = 128
_NCH = _RPW // _CHUNK


def _gather_body(tab_hbm, idx_hbm, out_hbm, idxv, rows, sem):
    wid = lax.axis_index("s") * _NC + lax.axis_index("c")
    base = wid * _RPW

    def chunk(c, _):
        off = base + c * _CHUNK
        pltpu.sync_copy(idx_hbm.at[pl.ds(off, _CHUNK)], idxv)
        pltpu.async_copy(tab_hbm.at[idxv], rows, sem).wait()
        pltpu.sync_copy(rows, out_hbm.at[pl.ds(off, _CHUNK)])
        return 0

    lax.fori_loop(0, _NCH, chunk, 0)


@functools.cache
def _get_gather():
  return pl.kernel(
    _gather_body,
    out_type=jax.ShapeDtypeStruct((_ROWS, _D), jnp.float32),
    mesh=plsc.VectorSubcoreMesh(core_axis_name="c", subcore_axis_name="s",
                                num_cores=_NC, num_subcores=_NS),
    compiler_params=pltpu.CompilerParams(needs_layout_passes=False),
    scratch_types=[
        pltpu.VMEM((_CHUNK,), jnp.int32),
        pltpu.VMEM((_CHUNK, _D), jnp.float32),
        pltpu.SemaphoreType.DMA,
    ],
  )


def kernel(xyz, feat):
    B, N, _ = xyz.shape
    initf = jax.random.randint(jax.random.key(42), (B,), 0, N)
    farth = initf.astype(jnp.int32).reshape(B, 1)
    dist = jnp.full((B, N), 1e10, jnp.float32)
    xyz_cbn = jnp.transpose(xyz, (2, 0, 1))
    x_all = xyz_cbn[0].reshape(B * N)
    y_all = xyz_cbn[1].reshape(B * N)
    z_all = xyz_cbn[2].reshape(B * N)
    ballq = _get_ballq()

    def run_ballq(sxyz_p):
        return ballq(x_all, y_all, z_all,
                     sxyz_p[0].reshape(B * _SC),
                     sxyz_p[1].reshape(B * _SC),
                     sxyz_p[2].reshape(B * _SC)).reshape(B, _SC, _K)

    sx_chunks = []
    nbr_chunks = []
    prev = None
    for _p in range(_P):
        sxyz_p, dist, farth = _fps_chunk(xyz_cbn, dist, farth)
        sx_chunks.append(sxyz_p)
        if prev is not None:
            nbr_chunks.append(run_ballq(prev))
        prev = sxyz_p
    nbr_chunks.append(run_ballq(prev))
    sxyz = jnp.concatenate(sx_chunks, axis=2)
    sample_xyz = jnp.transpose(sxyz, (1, 2, 0))
    nbr = jnp.concatenate(nbr_chunks, axis=1)
    tab = feat.reshape(B * N, _D)
    grouped = _get_gather()(tab, nbr.reshape(_ROWS))
    feat_group = grouped.reshape(B, _S, _K, _D)
    return sample_xyz, feat_group


# speedup vs baseline: 1.0660x; 1.0660x over previous
import functools

import jax
import jax.numpy as jnp
from jax import lax
from jax.experimental import pallas as pl
from jax.experimental.pallas import tpu as pltpu
from jax.experimental.pallas import tpu_sc as plsc

_B = 8
_N = 4096
_D = 256
_S = 512
_K = 32
_R2 = 0.2 ** 2

_NC = 2
_NS = 16
_NW = _NC * _NS
_L = 16


def _fps_body(xyz_ref, initf_ref, out_ref):
    x = xyz_ref[0]
    y = xyz_ref[1]
    z = xyz_ref[2]
    lane_iota = lax.broadcasted_iota(jnp.int32, (_B, _N), 1)
    s_iota = lax.broadcasted_iota(jnp.int32, (_B, _S), 1)

    def body(k, carry):
        distance, farth, sx, sy, sz = carry
        onehot = lane_iota == farth
        cx = jnp.sum(jnp.where(onehot, x, 0.0), axis=1, keepdims=True)
        cy = jnp.sum(jnp.where(onehot, y, 0.0), axis=1, keepdims=True)
        cz = jnp.sum(jnp.where(onehot, z, 0.0), axis=1, keepdims=True)
        sel = s_iota == k
        sx = jnp.where(sel, cx, sx)
        sy = jnp.where(sel, cy, sy)
        sz = jnp.where(sel, cz, sz)
        dx = x - cx
        dy = y - cy
        dz = z - cz
        dist = dx * dx + dy * dy + dz * dz
        distance = jnp.minimum(dist, distance)
        m = jnp.max(distance, axis=1, keepdims=True)
        cand = jnp.where(distance == m, lane_iota, _N)
        farth = jnp.min(cand, axis=1, keepdims=True)
        return distance, farth, sx, sy, sz

    init = (
        jnp.full((_B, _N), 1e10, jnp.float32),
        initf_ref[...],
        jnp.zeros((_B, _S), jnp.float32),
        jnp.zeros((_B, _S), jnp.float32),
        jnp.zeros((_B, _S), jnp.float32),
    )
    _, _, sx, sy, sz = lax.fori_loop(0, _S, body, init)
    out_ref[0] = sx
    out_ref[1] = sy
    out_ref[2] = sz


_fps = pl.pallas_call(
    _fps_body,
    out_shape=jax.ShapeDtypeStruct((3, _B, _S), jnp.float32),
)


_SBLK = _S // 4
_NCHUNK = _N // _L


_CBLK = 8


def _ballq_body(x_hbm, y_hbm, z_hbm, sx_hbm, sy_hbm, sz_hbm, out_hbm,
                xv, yv, zv, sxv, syv, szv, idxbuf, outv):
    wid = lax.axis_index("s") * _NC + lax.axis_index("c")
    b = wid // 4
    q = wid % 4
    pltpu.sync_copy(x_hbm.at[pl.ds(b * _N, _N)], xv)
    pltpu.sync_copy(y_hbm.at[pl.ds(b * _N, _N)], yv)
    pltpu.sync_copy(z_hbm.at[pl.ds(b * _N, _N)], zv)
    soff = b * _S + q * _SBLK
    pltpu.sync_copy(sx_hbm.at[pl.ds(soff, _SBLK)], sxv)
    pltpu.sync_copy(sy_hbm.at[pl.ds(soff, _SBLK)], syv)
    pltpu.sync_copy(sz_hbm.at[pl.ds(soff, _SBLK)], szv)

    r2 = jnp.float32(_R2)
    base_flat = b * _N
    i16 = lax.iota(jnp.int32, 16)

    def per_chunk16(t, _):
        svx = sxv[pl.ds(t * _L, _L)]
        svy = syv[pl.ds(t * _L, _L)]
        svz = szv[pl.ds(t * _L, _L)]
        for j in range(_L):
            i = t * _L + j
            sx = svx[j]
            sy = svy[j]
            sz = svz[j]

            def do_blk(bi, cnt):
                for cc in range(_CBLK):
                    c = bi * _CBLK + cc
                    px = xv[pl.ds(c * _L, _L)]
                    py = yv[pl.ds(c * _L, _L)]
                    pz = zv[pl.ds(c * _L, _L)]
                    dx = px - sx
                    dy = py - sy
                    dz = pz - sz
                    d = (dx * dx + dy * dy) + dz * dz
                    msk = d <= r2
                    ivec = i16 + c * _L
                    plsc.store_compressed(idxbuf.at[pl.ds(cnt, _L)],
                                          ivec, mask=msk)
                    cnt = cnt + plsc.all_reduce_population_count(msk)[0]
                return cnt

            def blk(bi, cnt):
                return lax.cond(cnt < _K, do_blk, lambda _, c: c, bi, cnt)

            cnt = lax.fori_loop(0, _NCHUNK // _CBLK, blk, jnp.int32(0))
            v0 = idxbuf[pl.ds(0, _L)]
            v1 = idxbuf[pl.ds(_L, _L)]
            first = v0[0]
            o0 = jnp.where(i16 < cnt, v0, first) + base_flat
            o1 = jnp.where((i16 + _L) < cnt, v1, first) + base_flat
            outv[i, pl.ds(0, _L)] = o0
            outv[i, pl.ds(_L, _L)] = o1
        return 0

    lax.fori_loop(0, _SBLK // _L, per_chunk16, 0)
    pltpu.sync_copy(outv, out_hbm.at[pl.ds(b * _S + q * _SBLK, _SBLK)])


@functools.cache
def _get_ballq():
  return pl.kernel(
    _ballq_body,
    out_type=jax.ShapeDtypeStruct((_B * _S, _K), jnp.int32),
    mesh=plsc.VectorSubcoreMesh(core_axis_name="c", subcore_axis_name="s",
                                num_cores=_NC, num_subcores=_NS),
    compiler_params=pltpu.CompilerParams(needs_layout_passes=False),
    scratch_types=[
        pltpu.VMEM((_N,), jnp.float32),
        pltpu.VMEM((_N,), jnp.float32),
        pltpu.VMEM((_N,), jnp.float32),
        pltpu.VMEM((_SBLK,), jnp.float32),
        pltpu.VMEM((_SBLK,), jnp.float32),
        pltpu.VMEM((_SBLK,), jnp.float32),
        pltpu.VMEM((160,), jnp.int32),
        pltpu.VMEM((_SBLK, _K), jnp.int32),
    ],
  )


_ROWS = _B * _S * _K
_RPW = _ROWS // _NW
_CHUNK = 128
_NCH = _RPW // _CHUNK


def _gather_body(tab_hbm, idx_hbm, out_hbm, idxv, rows, sem):
    wid = lax.axis_index("s") * _NC + lax.axis_index("c")
    base = wid * _RPW

    def chunk(c, _):
        off = base + c * _CHUNK
        pltpu.sync_copy(idx_hbm.at[pl.ds(off, _CHUNK)], idxv)
        pltpu.async_copy(tab_hbm.at[idxv], rows, sem).wait()
        pltpu.sync_copy(rows, out_hbm.at[pl.ds(off, _CHUNK)])
        return 0

    lax.fori_loop(0, _NCH, chunk, 0)


@functools.cache
def _get_gather():
  return pl.kernel(
    _gather_body,
    out_type=jax.ShapeDtypeStruct((_ROWS, _D), jnp.float32),
    mesh=plsc.VectorSubcoreMesh(core_axis_name="c", subcore_axis_name="s",
                                num_cores=_NC, num_subcores=_NS),
    compiler_params=pltpu.CompilerParams(needs_layout_passes=False),
    scratch_types=[
        pltpu.VMEM((_CHUNK,), jnp.int32),
        pltpu.VMEM((_CHUNK, _D), jnp.float32),
        pltpu.SemaphoreType.DMA,
    ],
  )


_DIAG = 0


def _xla_ballq(xyz, sample_xyz):
    N = xyz.shape[1]
    gi = jnp.broadcast_to(jnp.arange(N, dtype=jnp.int32)[None, None, :],
                          (xyz.shape[0], sample_xyz.shape[1], N))
    diff = sample_xyz[:, :, None, :] - xyz[:, None, :, :]
    sq = jnp.sum(diff ** 2, axis=-1)
    gi = jnp.where(sq > 0.2 ** 2, N, gi)
    gi = jnp.sort(gi, axis=-1)[:, :, :_K]
    gf = jnp.broadcast_to(gi[:, :, 0:1], gi.shape)
    return jnp.where(gi == N, gf, gi)


def kernel(xyz, feat):
    B, N, _ = xyz.shape
    initf = jax.random.randint(jax.random.key(42), (B,), 0, N)
    initf = initf.astype(jnp.int32).reshape(B, 1)
    xyz_cbn = jnp.transpose(xyz, (2, 0, 1))
    sxyz = _fps(xyz_cbn, initf)
    x_all = xyz_cbn[0].reshape(B * N)
    y_all = xyz_cbn[1].reshape(B * N)
    z_all = xyz_cbn[2].reshape(B * N)
    sx_all = sxyz[0].reshape(B * _S)
    sy_all = sxyz[1].reshape(B * _S)
    sz_all = sxyz[2].reshape(B * _S)
    sample_xyz = jnp.transpose(sxyz, (1, 2, 0))
    tab = feat.reshape(B * N, _D)
    if _DIAG == 1:
        nbr3 = _xla_ballq(xyz, sample_xyz)
        flat = nbr3 + jnp.arange(B, dtype=jnp.int32)[:, None, None] * N
        grouped = tab[flat.reshape(-1)]
        return sample_xyz, grouped.reshape(B, _S, _K, _D)
    nbr = _get_ballq()(x_all, y_all, z_all, sx_all, sy_all, sz_all)
    if _DIAG == 2:
        grouped = tab[nbr.reshape(-1)]
        return sample_xyz, grouped.reshape(B, _S, _K, _D)
    grouped = _get_gather()(tab, nbr.reshape(_ROWS))
    feat_group = grouped.reshape(B, _S, _K, _D)
    return sample_xyz, feat_group
